# Initial kernel scaffold; baseline (speedup 1.0000x reference)
import jax
import jax.numpy as jnp
from jax.experimental import pallas as pl


def kernel(edge_index, edge_weight, emb_node, emb_attri, W1, b1, W2, b2):
    raise NotImplementedError("write your pallas kernel here")


# speedup vs baseline: 13.3419x; 13.3419x over previous
import functools

import jax
import jax.numpy as jnp
from jax import lax
from jax.experimental import pallas as pl
from jax.experimental.pallas import tpu as pltpu
from jax.experimental.pallas import tpu_sc as plsc

_NNODE = 10000
_NATTRI = 6000
_NFEAT = 128
_NHID = 64
_NHID2 = 32
_E = 512000
_N = _NNODE + _NATTRI

_NW = 32
_CH = 128
_EPW = _E // _NW
_NCHUNK = _EPW // _CH
_ZR = _N // 16
_BM = 2000


def _make_spmm(D):
    mesh = plsc.VectorSubcoreMesh(core_axis_name="c", subcore_axis_name="s")

    @functools.partial(
        pl.kernel,
        out_type=jax.ShapeDtypeStruct((2, _N, D), jnp.float32),
        mesh=mesh,
        compiler_params=pltpu.CompilerParams(use_tc_tiling_on_sc=False),
        scratch_types=[
            pltpu.VMEM((_NCHUNK, _CH), jnp.int32),
            pltpu.VMEM((_NCHUNK, _CH), jnp.int32),
            pltpu.VMEM((_NCHUNK, _CH), jnp.float32),
            pltpu.VMEM((_CH, D), jnp.float32),
            pltpu.VMEM_SHARED((_N, D), jnp.float32),
            pltpu.SemaphoreType.DMA,
        ],
    )
    def spmm(src_hbm, dst_hbm, w_hbm, sup_hbm, zero_hbm, out_hbm,
             src_v, dst_v, w_v, rows_v, acc, gsem):
        cid = lax.axis_index("c")
        sid = lax.axis_index("s")
        wid = sid * 2 + cid

        pltpu.sync_copy(zero_hbm.at[pl.ds(sid * _ZR, _ZR)],
                        acc.at[pl.ds(sid * _ZR, _ZR)])
        pltpu.sync_copy(src_hbm.at[wid], src_v)
        pltpu.sync_copy(dst_hbm.at[wid], dst_v)
        pltpu.sync_copy(w_hbm.at[wid], w_v)
        plsc.subcore_barrier()

        def chunk(j, carry):
            pltpu.async_copy(sup_hbm.at[src_v.at[j]], rows_v, gsem).wait()

            @plsc.parallel_loop(0, _CH // 16, 1)
            def scale(g):
                w16 = w_v[j, pl.ds(g * 16, 16)]
                for l in range(16):
                    e = g * 16 + l
                    wsc = w16[l]
                    for q in range(D // 16):
                        sl = pl.ds(q * 16, 16)
                        rows_v[e, sl] = rows_v[e, sl] * wsc

            pltpu.sync_copy(rows_v, acc.at[dst_v.at[j]], add=True)
            return carry

        lax.fori_loop(0, _NCHUNK, chunk, 0)

        plsc.subcore_barrier()
        pltpu.sync_copy(acc.at[pl.ds(sid * _ZR, _ZR)],
                        out_hbm.at[cid, pl.ds(sid * _ZR, _ZR)])

    return spmm


def _mm1(x, W1):
    def body(x_ref, w_ref, o_ref):
        o_ref[...] = jnp.dot(x_ref[...], w_ref[...],
                             preferred_element_type=jnp.float32)

    return pl.pallas_call(
        body,
        grid=(_N // _BM,),
        in_specs=[pl.BlockSpec((_BM, _NFEAT), lambda i: (i, 0)),
                  pl.BlockSpec((_NFEAT, _NHID), lambda i: (0, 0))],
        out_specs=pl.BlockSpec((_BM, _NHID), lambda i: (i, 0)),
        out_shape=jax.ShapeDtypeStruct((_N, _NHID), jnp.float32),
    )(x, W1)


def _fuse1(parts, b1, W2):
    def body(p_ref, b_ref, w_ref, o_ref):
        h = jnp.maximum(p_ref[0] + p_ref[1] + b_ref[...], 0.0)
        o_ref[...] = jnp.dot(h, w_ref[...], preferred_element_type=jnp.float32)

    return pl.pallas_call(
        body,
        grid=(_N // _BM,),
        in_specs=[pl.BlockSpec((2, _BM, _NHID), lambda i: (0, i, 0)),
                  pl.BlockSpec((1, _NHID), lambda i: (0, 0)),
                  pl.BlockSpec((_NHID, _NHID2), lambda i: (0, 0))],
        out_specs=pl.BlockSpec((_BM, _NHID2), lambda i: (i, 0)),
        out_shape=jax.ShapeDtypeStruct((_N, _NHID2), jnp.float32),
    )(parts, b1.reshape(1, _NHID), W2)


def _fuse2(parts, b2):
    def body(p_ref, b_ref, o_ref):
        o_ref[...] = jnp.maximum(p_ref[0] + p_ref[1] + b_ref[...], 0.0)

    return pl.pallas_call(
        body,
        grid=(_N // _BM,),
        in_specs=[pl.BlockSpec((2, _BM, _NHID2), lambda i: (0, i, 0)),
                  pl.BlockSpec((1, _NHID2), lambda i: (0, 0))],
        out_specs=pl.BlockSpec((_BM, _NHID2), lambda i: (i, 0)),
        out_shape=jax.ShapeDtypeStruct((_N, _NHID2), jnp.float32),
    )(parts, b2.reshape(1, _NHID2))


def kernel(edge_index, edge_weight, emb_node, emb_attri, W1, b1, W2, b2):
    dst = edge_index[0].astype(jnp.int32).reshape(_NW, _NCHUNK, _CH)
    src = edge_index[1].astype(jnp.int32).reshape(_NW, _NCHUNK, _CH)
    w = edge_weight.astype(jnp.float32).reshape(_NW, _NCHUNK, _CH)
    zeros64 = jnp.zeros((_N, _NHID), jnp.float32)
    zeros32 = jnp.zeros((_N, _NHID2), jnp.float32)

    x = jnp.concatenate([emb_node, emb_attri], axis=0)
    sup1 = _mm1(x, W1)
    part1 = _make_spmm(_NHID)(src, dst, w, sup1, zeros64)
    sup2 = _fuse1(part1, b1, W2)
    part2 = _make_spmm(_NHID2)(src, dst, w, sup2, zeros32)
    return _fuse2(part2, b2)


# speedup vs baseline: 21.8547x; 1.6381x over previous
import functools

import jax
import jax.numpy as jnp
from jax import lax
from jax.experimental import pallas as pl
from jax.experimental.pallas import tpu as pltpu
from jax.experimental.pallas import tpu_sc as plsc

_NNODE = 10000
_NATTRI = 6000
_NFEAT = 128
_NHID = 64
_NHID2 = 32
_E = 512000
_N = _NNODE + _NATTRI

_NW = 32
_CH = 128
_EPW = _E // _NW
_NCHUNK = _EPW // _CH
_ZR = _N // 16
_BM = 2000
_NBUF = 5
_PFI = 4
_PF = 2


def _make_spmm(D):
    mesh = plsc.VectorSubcoreMesh(core_axis_name="c", subcore_axis_name="s")

    @functools.partial(
        pl.kernel,
        out_type=jax.ShapeDtypeStruct((2, _N, D), jnp.float32),
        mesh=mesh,
        compiler_params=pltpu.CompilerParams(use_tc_tiling_on_sc=False),
        scratch_types=[
            pltpu.VMEM((_NBUF, 2, _CH), jnp.int32),
            pltpu.VMEM((_NBUF, _CH), jnp.float32),
            pltpu.VMEM((_NBUF, _CH, D), jnp.float32),
            pltpu.VMEM_SHARED((_N, D), jnp.float32),
            pltpu.SemaphoreType.DMA((_NBUF,)),
            pltpu.SemaphoreType.DMA((_NBUF,)),
            pltpu.SemaphoreType.DMA((_NBUF,)),
        ],
    )
    def spmm(edata_hbm, w_hbm, sup_hbm, zero_hbm, out_hbm,
             ebuf, wbuf, rows_v, acc, isem, gsem, ssem):
        cid = lax.axis_index("c")
        sid = lax.axis_index("s")
        wid = sid * 2 + cid

        pltpu.sync_copy(zero_hbm.at[pl.ds(sid * _ZR, _ZR)],
                        acc.at[pl.ds(sid * _ZR, _ZR)])
        plsc.subcore_barrier()

        def idx_start(f, bf):
            pltpu.async_copy(edata_hbm.at[wid, f], ebuf.at[bf], isem.at[bf])
            pltpu.async_copy(w_hbm.at[wid, f], wbuf.at[bf], isem.at[bf])

        def idx_wait(f, bf):
            pltpu.make_async_copy(edata_hbm.at[wid, f], ebuf.at[bf],
                                  isem.at[bf]).wait()
            pltpu.make_async_copy(w_hbm.at[wid, f], wbuf.at[bf],
                                  isem.at[bf]).wait()

        def gather_start(f, bf):
            pltpu.async_copy(sup_hbm.at[ebuf.at[bf, 0]], rows_v.at[bf],
                             gsem.at[bf])

        def gather_wait(f, bf):
            pltpu.make_async_copy(sup_hbm.at[ebuf.at[bf, 0]], rows_v.at[bf],
                                  gsem.at[bf]).wait()

        def scatter_start(f, bf):
            pltpu.async_copy(rows_v.at[bf], acc.at[ebuf.at[bf, 1]],
                             ssem.at[bf], add=True)

        def scatter_wait(f, bf):
            pltpu.make_async_copy(rows_v.at[bf], acc.at[ebuf.at[bf, 1]],
                                  ssem.at[bf]).wait()

        for f in range(_PFI):
            idx_start(f, f % _NBUF)
        for f in range(_PF):
            idx_wait(f, f % _NBUF)
            gather_start(f, f % _NBUF)

        def chunk(t, carry):
            b = t % _NBUF

            fi = t + _PFI

            @pl.when(fi < _NCHUNK)
            def _():
                b2 = fi % _NBUF

                @pl.when(fi >= _NBUF)
                def _():
                    scatter_wait(fi - _NBUF, b2)

                idx_start(fi, b2)

            f = t + _PF

            @pl.when(f < _NCHUNK)
            def _():
                bf = f % _NBUF
                idx_wait(f, bf)
                gather_start(f, bf)

            gather_wait(t, b)

            @plsc.parallel_loop(0, _CH // 16, 1)
            def scale(g):
                w16 = wbuf[b, pl.ds(g * 16, 16)]
                for l in range(16):
                    e = g * 16 + l
                    wsc = w16[l]
                    for q in range(D // 16):
                        sl = pl.ds(q * 16, 16)
                        rows_v[b, e, sl] = rows_v[b, e, sl] * wsc

            scatter_start(t, b)
            return carry

        lax.fori_loop(0, _NCHUNK, chunk, 0)
        for i in range(_NBUF):
            f = _NCHUNK - _NBUF + i
            scatter_wait(f, f % _NBUF)

        plsc.subcore_barrier()
        pltpu.sync_copy(acc.at[pl.ds(sid * _ZR, _ZR)],
                        out_hbm.at[cid, pl.ds(sid * _ZR, _ZR)])

    return spmm


def _mm1(x, W1):
    def body(x_ref, w_ref, o_ref):
        o_ref[...] = jnp.dot(x_ref[...], w_ref[...],
                             preferred_element_type=jnp.float32)

    return pl.pallas_call(
        body,
        grid=(_N // _BM,),
        in_specs=[pl.BlockSpec((_BM, _NFEAT), lambda i: (i, 0)),
                  pl.BlockSpec((_NFEAT, _NHID), lambda i: (0, 0))],
        out_specs=pl.BlockSpec((_BM, _NHID), lambda i: (i, 0)),
        out_shape=jax.ShapeDtypeStruct((_N, _NHID), jnp.float32),
    )(x, W1)


def _fuse1(parts, b1, W2):
    def body(p_ref, b_ref, w_ref, o_ref):
        h = jnp.maximum(p_ref[0] + p_ref[1] + b_ref[...], 0.0)
        o_ref[...] = jnp.dot(h, w_ref[...], preferred_element_type=jnp.float32)

    return pl.pallas_call(
        body,
        grid=(_N // _BM,),
        in_specs=[pl.BlockSpec((2, _BM, _NHID), lambda i: (0, i, 0)),
                  pl.BlockSpec((1, _NHID), lambda i: (0, 0)),
                  pl.BlockSpec((_NHID, _NHID2), lambda i: (0, 0))],
        out_specs=pl.BlockSpec((_BM, _NHID2), lambda i: (i, 0)),
        out_shape=jax.ShapeDtypeStruct((_N, _NHID2), jnp.float32),
    )(parts, b1.reshape(1, _NHID), W2)


def _fuse2(parts, b2):
    def body(p_ref, b_ref, o_ref):
        o_ref[...] = jnp.maximum(p_ref[0] + p_ref[1] + b_ref[...], 0.0)

    return pl.pallas_call(
        body,
        grid=(_N // _BM,),
        in_specs=[pl.BlockSpec((2, _BM, _NHID2), lambda i: (0, i, 0)),
                  pl.BlockSpec((1, _NHID2), lambda i: (0, 0))],
        out_specs=pl.BlockSpec((_BM, _NHID2), lambda i: (i, 0)),
        out_shape=jax.ShapeDtypeStruct((_N, _NHID2), jnp.float32),
    )(parts, b2.reshape(1, _NHID2))


def kernel(edge_index, edge_weight, emb_node, emb_attri, W1, b1, W2, b2):
    dst = edge_index[0].astype(jnp.int32).reshape(_NW, _NCHUNK, _CH)
    src = edge_index[1].astype(jnp.int32).reshape(_NW, _NCHUNK, _CH)
    w = edge_weight.astype(jnp.float32).reshape(_NW, _NCHUNK, _CH)
    edata = jnp.stack([src, dst], axis=2)
    zeros64 = jnp.zeros((_N, _NHID), jnp.float32)
    zeros32 = jnp.zeros((_N, _NHID2), jnp.float32)

    x = jnp.concatenate([emb_node, emb_attri], axis=0)
    sup1 = _mm1(x, W1)
    part1 = _make_spmm(_NHID)(edata, w, sup1, zeros64)
    sup2 = _fuse1(part1, b1, W2)
    part2 = _make_spmm(_NHID2)(edata, w, sup2, zeros32)
    return _fuse2(part2, b2)


# speedup vs baseline: 23.2193x; 1.0624x over previous
import functools

import jax
import jax.numpy as jnp
from jax import lax
from jax.experimental import pallas as pl
from jax.experimental.pallas import tpu as pltpu
from jax.experimental.pallas import tpu_sc as plsc

_NNODE = 10000
_NATTRI = 6000
_NFEAT = 128
_NHID = 64
_NHID2 = 32
_E = 512000
_N = _NNODE + _NATTRI

_NW = 32
_CH = 128
_EPW = _E // _NW
_NCHUNK = _EPW // _CH
_ZR = _N // 16
_BM = 2000
_NBUF = 5
_PFI = 4
_PF = 2


def _make_spmm(D):
    mesh = plsc.VectorSubcoreMesh(core_axis_name="c", subcore_axis_name="s")

    @functools.partial(
        pl.kernel,
        out_type=jax.ShapeDtypeStruct((2, _N, D), jnp.float32),
        mesh=mesh,
        compiler_params=pltpu.CompilerParams(use_tc_tiling_on_sc=False),
        scratch_types=[
            pltpu.VMEM((_NBUF, 2, _CH), jnp.int32),
            pltpu.VMEM((_NBUF, _CH), jnp.float32),
            pltpu.VMEM((_NBUF, _CH, D), jnp.float32),
            pltpu.VMEM_SHARED((_N, D), jnp.float32),
            pltpu.SemaphoreType.DMA((_NBUF,)),
            pltpu.SemaphoreType.DMA((_NBUF,)),
            pltpu.SemaphoreType.DMA((_NBUF,)),
        ],
    )
    def spmm(edata_hbm, w_hbm, sup_hbm, out_hbm,
             ebuf, wbuf, rows_v, acc, isem, gsem, ssem):
        cid = lax.axis_index("c")
        sid = lax.axis_index("s")
        wid = sid * 2 + cid

        zvec = jnp.zeros((16,), jnp.float32)

        def zrow(e, carry):
            for q in range(D // 16):
                rows_v[0, e, pl.ds(q * 16, 16)] = zvec
            return carry

        lax.fori_loop(0, _CH, zrow, 0)
        for k in range(8):
            pltpu.sync_copy(rows_v.at[0].at[pl.ds(0, _ZR // 8)],
                            acc.at[pl.ds(sid * _ZR + k * (_ZR // 8), _ZR // 8)])
        plsc.subcore_barrier()

        def idx_start(f, bf):
            pltpu.async_copy(edata_hbm.at[wid, f], ebuf.at[bf], isem.at[bf])
            pltpu.async_copy(w_hbm.at[wid, f], wbuf.at[bf], isem.at[bf])

        def idx_wait(f, bf):
            pltpu.make_async_copy(edata_hbm.at[wid, f], ebuf.at[bf],
                                  isem.at[bf]).wait()
            pltpu.make_async_copy(w_hbm.at[wid, f], wbuf.at[bf],
                                  isem.at[bf]).wait()

        def gather_start(f, bf):
            pltpu.async_copy(sup_hbm.at[ebuf.at[bf, 0]], rows_v.at[bf],
                             gsem.at[bf])

        def gather_wait(f, bf):
            pltpu.make_async_copy(sup_hbm.at[ebuf.at[bf, 0]], rows_v.at[bf],
                                  gsem.at[bf]).wait()

        def scatter_start(f, bf):
            pltpu.async_copy(rows_v.at[bf], acc.at[ebuf.at[bf, 1]],
                             ssem.at[bf], add=True)

        def scatter_wait(f, bf):
            pltpu.make_async_copy(rows_v.at[bf], acc.at[ebuf.at[bf, 1]],
                                  ssem.at[bf]).wait()

        for f in range(_PFI):
            idx_start(f, f % _NBUF)
        for f in range(_PF):
            idx_wait(f, f % _NBUF)
            gather_start(f, f % _NBUF)

        def chunk(t, carry):
            b = t % _NBUF

            fi = t + _PFI

            @pl.when(fi < _NCHUNK)
            def _():
                b2 = fi % _NBUF

                @pl.when(fi >= _NBUF)
                def _():
                    scatter_wait(fi - _NBUF, b2)

                idx_start(fi, b2)

            f = t + _PF

            @pl.when(f < _NCHUNK)
            def _():
                bf = f % _NBUF
                idx_wait(f, bf)
                gather_start(f, bf)

            gather_wait(t, b)

            for g in range(_CH // 16):
                w16 = wbuf[b, pl.ds(g * 16, 16)]
                for l in range(16):
                    e = g * 16 + l
                    wsc = w16[l]
                    for q in range(D // 16):
                        sl = pl.ds(q * 16, 16)
                        rows_v[b, e, sl] = rows_v[b, e, sl] * wsc

            scatter_start(t, b)
            return carry

        lax.fori_loop(0, _NCHUNK, chunk, 0)
        for i in range(_NBUF):
            f = _NCHUNK - _NBUF + i
            scatter_wait(f, f % _NBUF)

        plsc.subcore_barrier()
        pltpu.sync_copy(acc.at[pl.ds(sid * _ZR, _ZR)],
                        out_hbm.at[cid, pl.ds(sid * _ZR, _ZR)])

    return spmm


def _mm1(x, W1):
    def body(x_ref, w_ref, o_ref):
        o_ref[...] = jnp.dot(x_ref[...], w_ref[...],
                             preferred_element_type=jnp.float32)

    return pl.pallas_call(
        body,
        grid=(_N // _BM,),
        in_specs=[pl.BlockSpec((_BM, _NFEAT), lambda i: (i, 0)),
                  pl.BlockSpec((_NFEAT, _NHID), lambda i: (0, 0))],
        out_specs=pl.BlockSpec((_BM, _NHID), lambda i: (i, 0)),
        out_shape=jax.ShapeDtypeStruct((_N, _NHID), jnp.float32),
    )(x, W1)


def _fuse1(parts, b1, W2):
    def body(p_ref, b_ref, w_ref, o_ref):
        h = jnp.maximum(p_ref[0] + p_ref[1] + b_ref[...], 0.0)
        o_ref[...] = jnp.dot(h, w_ref[...], preferred_element_type=jnp.float32)

    return pl.pallas_call(
        body,
        grid=(_N // _BM,),
        in_specs=[pl.BlockSpec((2, _BM, _NHID), lambda i: (0, i, 0)),
                  pl.BlockSpec((1, _NHID), lambda i: (0, 0)),
                  pl.BlockSpec((_NHID, _NHID2), lambda i: (0, 0))],
        out_specs=pl.BlockSpec((_BM, _NHID2), lambda i: (i, 0)),
        out_shape=jax.ShapeDtypeStruct((_N, _NHID2), jnp.float32),
    )(parts, b1.reshape(1, _NHID), W2)


def _fuse2(parts, b2):
    def body(p_ref, b_ref, o_ref):
        o_ref[...] = jnp.maximum(p_ref[0] + p_ref[1] + b_ref[...], 0.0)

    return pl.pallas_call(
        body,
        grid=(_N // _BM,),
        in_specs=[pl.BlockSpec((2, _BM, _NHID2), lambda i: (0, i, 0)),
                  pl.BlockSpec((1, _NHID2), lambda i: (0, 0))],
        out_specs=pl.BlockSpec((_BM, _NHID2), lambda i: (i, 0)),
        out_shape=jax.ShapeDtypeStruct((_N, _NHID2), jnp.float32),
    )(parts, b2.reshape(1, _NHID2))


def kernel(edge_index, edge_weight, emb_node, emb_attri, W1, b1, W2, b2):
    dst = edge_index[0].astype(jnp.int32).reshape(_NW, _NCHUNK, _CH)
    src = edge_index[1].astype(jnp.int32).reshape(_NW, _NCHUNK, _CH)
    w = edge_weight.astype(jnp.float32).reshape(_NW, _NCHUNK, _CH)
    edata = jnp.stack([src, dst], axis=2)

    x = jnp.concatenate([emb_node, emb_attri], axis=0)
    sup1 = _mm1(x, W1)
    part1 = _make_spmm(_NHID)(edata, w, sup1)
    sup2 = _fuse1(part1, b1, W2)
    part2 = _make_spmm(_NHID2)(edata, w, sup2)
    return _fuse2(part2, b2)
